# baseline (speedup 1.0000x reference)
import functools

import jax
import jax.numpy as jnp
from jax import lax
from jax.experimental import pallas as pl
from jax.experimental.pallas import tpu as pltpu
from jax.experimental.pallas import tpu_sc as plsc

B = 16384
D = 16
NF = 10
MLP_IN = (NF + 1) * D

NC = 2
NS = 16
NW = NC * NS

N_IDX = B * NF
N_PER_W = N_IDX // NW
CHUNK = 128
N_CHUNKS = N_PER_W // CHUNK


def _sc_gather(emb_hbm, fc_hbm, idx_hbm, emb_out, fc_out,
               idx_v, emb_v, fc_v, sem_e, sem_f):
    wid = lax.axis_index("s") * NC + lax.axis_index("c")
    base = wid * N_PER_W
    pltpu.sync_copy(idx_hbm.at[pl.ds(base, N_PER_W)], idx_v)

    def fire(j, carry):
        sl = pl.ds(j * CHUNK, CHUNK)
        pltpu.make_async_copy(emb_hbm.at[idx_v.at[sl]], emb_v.at[sl], sem_e).start()
        pltpu.make_async_copy(fc_hbm.at[idx_v.at[sl]], fc_v.at[sl, :], sem_f).start()
        return carry

    lax.fori_loop(0, N_CHUNKS, fire, 0)

    def drain(j, carry):
        sl = pl.ds(j * CHUNK, CHUNK)
        pltpu.make_async_copy(emb_hbm.at[idx_v.at[sl]], emb_v.at[sl], sem_e).wait()
        pltpu.make_async_copy(fc_hbm.at[idx_v.at[sl]], fc_v.at[sl, :], sem_f).wait()
        return carry

    lax.fori_loop(0, N_CHUNKS, drain, 0)

    pltpu.sync_copy(emb_v, emb_out.at[pl.ds(base, N_PER_W)])
    pltpu.sync_copy(fc_v, fc_out.at[pl.ds(base, N_PER_W), :])


@functools.cache
def _gather_call():
    return pl.kernel(
        _sc_gather,
        out_type=(
            jax.ShapeDtypeStruct((N_IDX, D), jnp.float32),
            jax.ShapeDtypeStruct((N_IDX, 1), jnp.float32),
        ),
        mesh=plsc.VectorSubcoreMesh(core_axis_name="c", subcore_axis_name="s"),
        scratch_types=[
            pltpu.VMEM((N_PER_W,), jnp.int32),
            pltpu.VMEM((N_PER_W, D), jnp.float32),
            pltpu.VMEM((N_PER_W, 1), jnp.float32),
            pltpu.SemaphoreType.DMA,
            pltpu.SemaphoreType.DMA,
        ],
        compiler_params=pltpu.CompilerParams(use_tc_tiling_on_sc=False),
    )


VOCAB = 1000000
TSLAB = 3200
NTS = VOCAB // TSLAB
TTAIL0 = NTS * TSLAB


def _tc_transpose(src_ref, out_ref):
    x = src_ref[...]
    z = x.T
    y = z.reshape(TSLAB // 8, 8, D)
    out_ref[...] = jnp.concatenate([y[:, k, :] for k in range(8)], axis=1)


@functools.cache
def _transpose_call():
    return pl.pallas_call(
        _tc_transpose,
        grid=(NTS,),
        in_specs=[pl.BlockSpec((D, TSLAB), lambda i: (0, i))],
        out_specs=pl.BlockSpec((TSLAB // 8, 128), lambda i: (i, 0)),
        out_shape=jax.ShapeDtypeStruct((VOCAB * D // 128, 128), jnp.float32),
    )


BB = 2048


def _tc_dense(emb_ref, fc_ref, genre_ref, bias_ref, wg_ref, w1_ref, b1_ref,
              w2_ref, b2_ref, w3_ref, b3_ref, out_ref):
    emb = emb_ref[...]
    genre = genre_ref[...]
    eg = jnp.dot(genre, wg_ref[...], preferred_element_type=jnp.float32)

    fields = [emb[:, f * D:(f + 1) * D] for f in range(NF)] + [eg]
    s = fields[0]
    sos = fields[0] * fields[0]
    for v in fields[1:]:
        s = s + v
        sos = sos + v * v
    fm2 = 0.5 * jnp.sum(s * s - sos, axis=1)

    fm1 = bias_ref[0] + jnp.sum(fc_ref[...], axis=1) + jnp.sum(eg, axis=1)

    h = jnp.concatenate([emb, eg], axis=1)
    h = jnp.dot(h, w1_ref[...], preferred_element_type=jnp.float32) + b1_ref[...]
    h = jnp.maximum(h, 0.0)
    h = jnp.dot(h, w2_ref[...], preferred_element_type=jnp.float32) + b2_ref[...]
    h = jnp.maximum(h, 0.0)
    mlp = jnp.dot(h, w3_ref[...], preferred_element_type=jnp.float32)[:, 0] + b3_ref[0]

    out_ref[...] = jax.nn.sigmoid(fm1 + fm2 + mlp)


@functools.cache
def _dense_call():
  return pl.pallas_call(
    _tc_dense,
    grid=(B // BB,),
    in_specs=[
        pl.BlockSpec((BB, NF * D), lambda i: (i, 0)),
        pl.BlockSpec((BB, NF), lambda i: (i, 0)),
        pl.BlockSpec((BB, 18), lambda i: (i, 0)),
        pl.BlockSpec(memory_space=pltpu.SMEM),
        pl.BlockSpec((18, D), lambda i: (0, 0)),
        pl.BlockSpec((MLP_IN, 128), lambda i: (0, 0)),
        pl.BlockSpec((128,), lambda i: (0,)),
        pl.BlockSpec((128, 64), lambda i: (0, 0)),
        pl.BlockSpec((64,), lambda i: (0,)),
        pl.BlockSpec((64, 1), lambda i: (0, 0)),
        pl.BlockSpec(memory_space=pltpu.SMEM),
    ],
    out_specs=pl.BlockSpec((BB,), lambda i: (i,)),
    out_shape=jax.ShapeDtypeStruct((B,), jnp.float32),
  )


def kernel(x, bias, fc_table, W_genre, emb_table, W1, b1, W2, b2, W3, b3):
    idx_flat = x[:, :NF].reshape(-1)
    genre = x[:, NF:].astype(jnp.float32)
    emb_lin = _transpose_call()(emb_table.T)
    tail = emb_table[TTAIL0:, :].reshape(TSLAB // 16, 128)
    emb_lin = lax.dynamic_update_slice(emb_lin, tail, (TTAIL0 * D // 128, 0))
    emb_lin = emb_lin.reshape(VOCAB, D)
    emb_g, fc_g = _gather_call()(emb_lin, fc_table, idx_flat)
    emb2 = emb_g.reshape(B, NF * D)
    fc2 = fc_g.reshape(B, NF)
    return _dense_call()(emb2, fc2, genre, bias, W_genre, W1, b1, W2, b2, W3, b3)


# speedup vs baseline: 2.8767x; 2.8767x over previous
import functools

import jax
import jax.numpy as jnp
from jax import lax
from jax.experimental import pallas as pl
from jax.experimental.pallas import tpu as pltpu
from jax.experimental.pallas import tpu_sc as plsc

B = 16384
D = 16
NF = 10
MLP_IN = (NF + 1) * D

NC = 2
NS = 16
NW = NC * NS

N_IDX = B * NF
N_PER_W = N_IDX // NW
CHUNK = 128
N_CHUNKS = N_PER_W // CHUNK


def _sc_gather(emb_hbm, fc_hbm, idx_hbm, emb_out, fc_out,
               idx_v, emb_v, fc_v, sem_e, sem_f):
    wid = lax.axis_index("s") * NC + lax.axis_index("c")
    base = wid * N_PER_W
    pltpu.sync_copy(idx_hbm.at[pl.ds(base, N_PER_W)], idx_v)

    def fire(j, carry):
        sl = pl.ds(j * CHUNK, CHUNK)
        pltpu.make_async_copy(emb_hbm.at[idx_v.at[sl]], emb_v.at[sl], sem_e).start()
        pltpu.make_async_copy(fc_hbm.at[idx_v.at[sl]], fc_v.at[sl], sem_f).start()
        return carry

    lax.fori_loop(0, N_CHUNKS, fire, 0)

    def drain(j, carry):
        sl = pl.ds(j * CHUNK, CHUNK)
        pltpu.make_async_copy(emb_hbm.at[idx_v.at[sl]], emb_v.at[sl], sem_e).wait()
        pltpu.make_async_copy(fc_hbm.at[idx_v.at[sl]], fc_v.at[sl], sem_f).wait()
        return carry

    lax.fori_loop(0, N_CHUNKS, drain, 0)

    pltpu.sync_copy(emb_v, emb_out.at[pl.ds(base, N_PER_W)])
    pltpu.sync_copy(fc_v, fc_out.at[pl.ds(base, N_PER_W)])


@functools.cache
def _gather_call():
    return pl.kernel(
        _sc_gather,
        out_type=(
            jax.ShapeDtypeStruct((N_IDX, D), jnp.float32),
            jax.ShapeDtypeStruct((N_IDX,), jnp.float32),
        ),
        mesh=plsc.VectorSubcoreMesh(core_axis_name="c", subcore_axis_name="s"),
        scratch_types=[
            pltpu.VMEM((N_PER_W,), jnp.int32),
            pltpu.VMEM((N_PER_W, D), jnp.float32),
            pltpu.VMEM((N_PER_W,), jnp.float32),
            pltpu.SemaphoreType.DMA,
            pltpu.SemaphoreType.DMA,
        ],
        compiler_params=pltpu.CompilerParams(use_tc_tiling_on_sc=False),
    )


VOCAB = 1000000
TSLAB = 3200
NTS = VOCAB // TSLAB
TTAIL0 = NTS * TSLAB


def _tc_transpose(src_ref, out_ref):
    x = src_ref[...]
    z = x.T
    y = z.reshape(TSLAB // 8, 8, D)
    out_ref[...] = jnp.concatenate([y[:, k, :] for k in range(8)], axis=1)


@functools.cache
def _transpose_call():
    return pl.pallas_call(
        _tc_transpose,
        grid=(NTS,),
        in_specs=[pl.BlockSpec((D, TSLAB), lambda i: (0, i))],
        out_specs=pl.BlockSpec((TSLAB // 8, 128), lambda i: (i, 0)),
        out_shape=jax.ShapeDtypeStruct((VOCAB * D // 128, 128), jnp.float32),
    )


BB = 2048


def _tc_dense(emb_ref, fc_ref, genre_ref, bias_ref, wg_ref, w1_ref, b1_ref,
              w2_ref, b2_ref, w3_ref, b3_ref, out_ref):
    emb = emb_ref[...]
    genre = genre_ref[...]
    eg = jnp.dot(genre, wg_ref[...], preferred_element_type=jnp.float32)

    sel = (lax.broadcasted_iota(jnp.int32, (NF * D, D), 0) % D
           == lax.broadcasted_iota(jnp.int32, (NF * D, D), 1)).astype(jnp.float32)
    t = jnp.dot(emb, sel, precision=lax.Precision.HIGHEST,
                preferred_element_type=jnp.float32)
    tsq = jnp.dot(emb * emb, sel, precision=lax.Precision.HIGHEST,
                  preferred_element_type=jnp.float32)
    s = t + eg
    sos = tsq + eg * eg
    fm2 = 0.5 * jnp.sum(s * s - sos, axis=1)

    fm1 = bias_ref[0] + jnp.sum(fc_ref[...], axis=1) + jnp.sum(eg, axis=1)

    h = jnp.concatenate([emb, eg], axis=1)
    h = jnp.dot(h, w1_ref[...], preferred_element_type=jnp.float32) + b1_ref[...]
    h = jnp.maximum(h, 0.0)
    h = jnp.dot(h, w2_ref[...], preferred_element_type=jnp.float32) + b2_ref[...]
    h = jnp.maximum(h, 0.0)
    mlp = jnp.dot(h, w3_ref[...], preferred_element_type=jnp.float32)[:, 0] + b3_ref[0]

    out_ref[...] = jax.nn.sigmoid(fm1 + fm2 + mlp)


@functools.cache
def _dense_call():
  return pl.pallas_call(
    _tc_dense,
    grid=(B // BB,),
    in_specs=[
        pl.BlockSpec((BB, NF * D), lambda i: (i, 0)),
        pl.BlockSpec((BB, NF), lambda i: (i, 0)),
        pl.BlockSpec((BB, 18), lambda i: (i, 0)),
        pl.BlockSpec(memory_space=pltpu.SMEM),
        pl.BlockSpec((18, D), lambda i: (0, 0)),
        pl.BlockSpec((MLP_IN, 128), lambda i: (0, 0)),
        pl.BlockSpec((128,), lambda i: (0,)),
        pl.BlockSpec((128, 64), lambda i: (0, 0)),
        pl.BlockSpec((64,), lambda i: (0,)),
        pl.BlockSpec((64, 1), lambda i: (0, 0)),
        pl.BlockSpec(memory_space=pltpu.SMEM),
    ],
    out_specs=pl.BlockSpec((BB,), lambda i: (i,)),
    out_shape=jax.ShapeDtypeStruct((B,), jnp.float32),
  )


def kernel(x, bias, fc_table, W_genre, emb_table, W1, b1, W2, b2, W3, b3):
    idx_flat = x[:, :NF].reshape(-1)
    genre = x[:, NF:].astype(jnp.float32)
    emb_lin = _transpose_call()(emb_table.T)
    tail = emb_table[TTAIL0:, :].reshape(TSLAB // 16, 128)
    emb_lin = lax.dynamic_update_slice(emb_lin, tail, (TTAIL0 * D // 128, 0))
    emb_lin = emb_lin.reshape(VOCAB, D)
    emb_g, fc_g = _gather_call()(emb_lin, fc_table.reshape(-1), idx_flat)
    emb2 = emb_g.reshape(B, NF * D)
    fc2 = fc_g.reshape(B, NF)
    return _dense_call()(emb2, fc2, genre, bias, W_genre, W1, b1, W2, b2, W3, b3)
